# baseline (speedup 1.0000x reference)
import functools

import jax
import jax.numpy as jnp
from jax import lax
from jax.experimental import pallas as pl
from jax.experimental.pallas import tpu as pltpu
from jax.experimental.pallas import tpu_sc as plsc

_NUM_WORKERS = 32
_TW = 128
_W_COLS = 244 * _TW
_C_COLS = 61 * _TW
_TAIL0 = _NUM_WORKERS * _W_COLS
_TAILW = 5 * _TW
_CAP = 2048
_L = 16
_D = 8


def _sweep_kernel(idx2d, table_t, cff_scales, B):
    mesh = plsc.VectorSubcoreMesh(core_axis_name="c", subcore_axis_name="s")

    @functools.partial(
        pl.kernel,
        mesh=mesh,
        compiler_params=pltpu.CompilerParams(needs_layout_passes=False),
        out_type=(
            jax.ShapeDtypeStruct((_NUM_WORKERS * _CAP * _D,), jnp.float32),
            jax.ShapeDtypeStruct((_NUM_WORKERS * _CAP,), jnp.int32),
        ),
        scratch_types=[
            pltpu.VMEM((B,), jnp.int32),
            pltpu.VMEM((_D, _C_COLS), jnp.float32),
            pltpu.VMEM((_CAP,), jnp.int32),
            pltpu.VMEM((_CAP,), jnp.int32),
            pltpu.VMEM((_D, _CAP), jnp.float32),
            pltpu.VMEM((_L,), jnp.float32),
            pltpu.SemaphoreType.DMA,
        ],
    )
    def k1(idx_hbm, table_hbm, scales_hbm, vals_hbm, bpos_hbm, idx_vm, buf,
           rlist, blist, vfm, sc_v, sem):
        wid = lax.axis_index("s") * 2 + lax.axis_index("c")
        pltpu.sync_copy(idx_hbm, idx_vm)
        pltpu.sync_copy(scales_hbm, sc_v.at[pl.ds(0, _D)])
        iota = lax.iota(jnp.int32, _L)
        lo = wid * _W_COLS
        lov = lax.broadcast_in_dim(lo, (_L,), ())
        hiv = lov + _W_COLS
        wz = lax.broadcast_in_dim(wid == 0, (_L,), ())

        def match(i, cnt):
            v = idx_vm[pl.ds(i * _L, _L)]
            m = jnp.logical_and(v >= lov, v < hiv)
            m = jnp.logical_or(m, jnp.logical_and(wz, v >= _TAIL0))
            plsc.store_compressed(rlist.at[pl.ds(cnt, _L)], v, mask=m)
            plsc.store_compressed(
                blist.at[pl.ds(cnt, _L)], i * _L + iota, mask=m
            )
            pc = plsc.all_reduce_population_count(m)
            return lax.min(cnt + pc[0], _CAP - _L)

        cnt = lax.fori_loop(0, B // _L, match, 0)
        trips = lax.div(cnt + _L - 1, _L)
        cntv = lax.broadcast_in_dim(cnt, (_L,), ())

        s = sc_v[...]
        sfeat = [lax.broadcast_in_dim(s[c], (_L,), ()) for c in range(_D)]

        def process_chunk(col0, width):
            off = pl.multiple_of(col0, _TW)
            copies = [
                pltpu.async_copy(
                    table_hbm.at[:, pl.ds(off + t * _TW, _TW)],
                    buf.at[:, pl.ds(t * _TW, _TW)],
                    sem,
                )
                for t in range(width // _TW)
            ]
            for cp in copies:
                cp.wait()
            col0v = lax.broadcast_in_dim(col0, (_L,), ())

            def scan(j, _):
                pos = j * _L + iota
                v = rlist[pl.ds(j * _L, _L)]
                m = jnp.logical_and(pos < cntv, v >= col0v)
                m = jnp.logical_and(m, v < col0v + width)
                local = v - col0v
                for c in range(_D):
                    cv = lax.broadcast_in_dim(c, (_L,), ())
                    vals = plsc.load_gather(buf, [cv, local], mask=m)
                    plsc.store_scatter(
                        vfm, [cv, pos], vals * sfeat[c], mask=m
                    )
                return 0

            lax.fori_loop(0, trips, scan, 0)

        for ch in range(_W_COLS // _C_COLS):
            process_chunk(lo + ch * _C_COLS, _C_COLS)
        process_chunk(lo * 0 + _TAIL0, _TAILW)

        dummyv = (
            lax.broadcast_in_dim(B + wid * 4, (_L,), ())
            + lax.bitwise_and(iota, 3)
        )

        def pad(i, _):
            pos = i * _L + iota
            keep = pos < cntv
            cur = blist[pl.ds(i * _L, _L)]
            blist[pl.ds(i * _L, _L)] = jnp.where(keep, cur, dummyv)
            return 0

        lax.fori_loop(0, _CAP // _L, pad, 0)

        for c in range(_D):
            pltpu.sync_copy(
                vfm.at[c], vals_hbm.at[pl.ds((wid * _D + c) * _CAP, _CAP)]
            )
        pltpu.sync_copy(blist, bpos_hbm.at[pl.ds(wid * _CAP, _CAP)])

    return k1(idx2d, table_t, cff_scales)


def _permute_kernel(vals1d, bpos1d, B):
    mesh = plsc.VectorSubcoreMesh(core_axis_name="c", subcore_axis_name="s")

    @functools.partial(
        pl.kernel,
        mesh=mesh,
        compiler_params=pltpu.CompilerParams(
            needs_layout_passes=False, use_tc_tiling_on_sc=False
        ),
        out_type=jax.ShapeDtypeStruct((B + 128, _D), jnp.float32),
        scratch_types=[
            pltpu.VMEM((_D, _CAP), jnp.float32),
            pltpu.VMEM((_CAP, _D), jnp.float32),
            pltpu.VMEM((_CAP,), jnp.int32),
            pltpu.VMEM((_CAP // 128, 128), jnp.int32),
            pltpu.SemaphoreType.DMA,
        ],
    )
    def k2(vals_hbm, bpos_hbm, out_hbm, vfm, vbm, bl1, bl2, sem):
        wid = lax.axis_index("s") * 2 + lax.axis_index("c")
        for c in range(_D):
            pltpu.sync_copy(
                vals_hbm.at[pl.ds((wid * _D + c) * _CAP, _CAP)], vfm.at[c]
            )
        pltpu.sync_copy(bpos_hbm.at[pl.ds(wid * _CAP, _CAP)], bl1)
        iota = lax.iota(jnp.int32, _L)

        def transpose(i, _):
            pos = i * _L + iota
            for c in range(_D):
                cv = lax.broadcast_in_dim(c, (_L,), ())
                vals = vfm[c, pl.ds(i * _L, _L)]
                plsc.store_scatter(vbm, [pos, cv], vals)
            return 0

        lax.fori_loop(0, _CAP // _L, transpose, 0)
        for row in range(_CAP // 128):
            for c16 in range(128 // _L):
                bl2[row, pl.ds(c16 * _L, _L)] = bl1[
                    pl.ds(row * 128 + c16 * _L, _L)
                ]
        for row in range(_CAP // 128):
            pltpu.sync_copy(
                vbm.at[pl.ds(row * 128, 128), :],
                out_hbm.at[bl2.at[row]],
            )

    return k2(vals1d, bpos1d)


def kernel(point_id, cffs_scaled, cff_scales):
    B = point_id.shape[0]
    table_t = cffs_scaled.T
    idx1d = point_id.astype(jnp.int32)
    vals1d, bpos1d = _sweep_kernel(idx1d, table_t, cff_scales, B)
    out = _permute_kernel(vals1d, bpos1d, B)
    return out[:B]


# speedup vs baseline: 1.4576x; 1.4576x over previous
import functools

import jax
import jax.numpy as jnp
from jax import lax
from jax.experimental import pallas as pl
from jax.experimental.pallas import tpu as pltpu
from jax.experimental.pallas import tpu_sc as plsc

_NUM_WORKERS = 32
_TW = 128
_W_COLS = 244 * _TW
_C_COLS = 61 * _TW
_TAIL0 = _NUM_WORKERS * _W_COLS
_TAILW = 5 * _TW
_CAP = 2048
_L = 16
_D = 8


def _sweep_kernel(idx2d, table_t, cff_scales, B):
    mesh = plsc.VectorSubcoreMesh(core_axis_name="c", subcore_axis_name="s")

    @functools.partial(
        pl.kernel,
        mesh=mesh,
        compiler_params=pltpu.CompilerParams(needs_layout_passes=False),
        out_type=(
            jax.ShapeDtypeStruct((_NUM_WORKERS * _CAP * _D,), jnp.float32),
            jax.ShapeDtypeStruct((_NUM_WORKERS * _CAP,), jnp.int32),
            jax.ShapeDtypeStruct((_NUM_WORKERS * _L,), jnp.int32),
        ),
        scratch_types=[
            pltpu.VMEM((B,), jnp.int32),
            pltpu.VMEM((_D, _C_COLS), jnp.float32),
            pltpu.VMEM((_CAP,), jnp.int32),
            pltpu.VMEM((_CAP,), jnp.int32),
            pltpu.VMEM((_D, _CAP), jnp.float32),
            pltpu.VMEM((_L,), jnp.float32),
            pltpu.VMEM((_L,), jnp.int32),
            pltpu.SemaphoreType.DMA,
        ],
    )
    def k1(idx_hbm, table_hbm, scales_hbm, vals_hbm, bpos_hbm, cnt_hbm,
           idx_vm, buf, rlist, blist, vfm, sc_v, cnt_v, sem):
        wid = lax.axis_index("s") * 2 + lax.axis_index("c")
        pltpu.sync_copy(idx_hbm, idx_vm)
        pltpu.sync_copy(scales_hbm, sc_v.at[pl.ds(0, _D)])
        iota = lax.iota(jnp.int32, _L)
        lo = wid * _W_COLS
        lov = lax.broadcast_in_dim(lo, (_L,), ())
        hiv = lov + _W_COLS
        wz = lax.broadcast_in_dim(wid == 0, (_L,), ())

        def match(i, cnt):
            v = idx_vm[pl.ds(i * _L, _L)]
            m = jnp.logical_and(v >= lov, v < hiv)
            m = jnp.logical_or(m, jnp.logical_and(wz, v >= _TAIL0))
            plsc.store_compressed(rlist.at[pl.ds(cnt, _L)], v, mask=m)
            plsc.store_compressed(
                blist.at[pl.ds(cnt, _L)], i * _L + iota, mask=m
            )
            pc = plsc.all_reduce_population_count(m)
            return lax.min(cnt + pc[0], _CAP - _L)

        cnt = lax.fori_loop(0, B // _L, match, 0)
        trips = lax.div(cnt + _L - 1, _L)
        cntv = lax.broadcast_in_dim(cnt, (_L,), ())

        s = sc_v[...]
        sfeat = [lax.broadcast_in_dim(s[c], (_L,), ()) for c in range(_D)]

        def process_chunk(col0, width):
            off = pl.multiple_of(col0, _TW)
            copies = [
                pltpu.async_copy(
                    table_hbm.at[:, pl.ds(off + t * _TW, _TW)],
                    buf.at[:, pl.ds(t * _TW, _TW)],
                    sem,
                )
                for t in range(width // _TW)
            ]
            for cp in copies:
                cp.wait()
            col0v = lax.broadcast_in_dim(col0, (_L,), ())

            def scan(j, _):
                pos = j * _L + iota
                v = rlist[pl.ds(j * _L, _L)]
                m = jnp.logical_and(pos < cntv, v >= col0v)
                m = jnp.logical_and(m, v < col0v + width)
                local = v - col0v
                for c in range(_D):
                    cv = lax.broadcast_in_dim(c, (_L,), ())
                    vals = plsc.load_gather(buf, [cv, local], mask=m)
                    plsc.store_scatter(
                        vfm, [cv, pos], vals * sfeat[c], mask=m
                    )
                return 0

            lax.fori_loop(0, trips, scan, 0)

        for ch in range(_W_COLS // _C_COLS):
            process_chunk(lo + ch * _C_COLS, _C_COLS)
        process_chunk(lo * 0 + _TAIL0, _TAILW)

        dummyv = (
            lax.broadcast_in_dim(B + wid * 4, (_L,), ())
            + lax.bitwise_and(iota, 3)
        )

        def pad(i, _):
            pos = i * _L + iota
            keep = pos < cntv
            cur = blist[pl.ds(i * _L, _L)]
            blist[pl.ds(i * _L, _L)] = jnp.where(keep, cur, dummyv)
            return 0

        lax.fori_loop(0, _CAP // _L, pad, 0)

        for c in range(_D):
            pltpu.sync_copy(
                vfm.at[c], vals_hbm.at[pl.ds((wid * _D + c) * _CAP, _CAP)]
            )
        pltpu.sync_copy(blist, bpos_hbm.at[pl.ds(wid * _CAP, _CAP)])
        cnt_v[pl.ds(0, _L)] = cntv
        pltpu.sync_copy(cnt_v, cnt_hbm.at[pl.ds(wid * _L, _L)])

    return k1(idx2d, table_t, cff_scales)


def _permute_kernel(vals1d, bpos1d, cnts, B):
    mesh = plsc.VectorSubcoreMesh(core_axis_name="c", subcore_axis_name="s")

    @functools.partial(
        pl.kernel,
        mesh=mesh,
        compiler_params=pltpu.CompilerParams(
            needs_layout_passes=False, use_tc_tiling_on_sc=False
        ),
        out_type=jax.ShapeDtypeStruct((B + 128, 2 * _D), jnp.float32),
        scratch_types=[
            pltpu.VMEM((_D, _CAP), jnp.float32),
            pltpu.VMEM((_CAP, 2 * _D), jnp.float32),
            pltpu.VMEM((_CAP,), jnp.int32),
            pltpu.VMEM((_CAP // 128, 128), jnp.int32),
            pltpu.VMEM((_L,), jnp.int32),
            pltpu.SemaphoreType.DMA,
        ],
    )
    def k2(vals_hbm, bpos_hbm, cnt_hbm, out_hbm, vfm, vbm, bl1, bl2,
           cnt_v, sem):
        wid = lax.axis_index("s") * 2 + lax.axis_index("c")
        for c in range(_D):
            pltpu.sync_copy(
                vals_hbm.at[pl.ds((wid * _D + c) * _CAP, _CAP)], vfm.at[c]
            )
        pltpu.sync_copy(bpos_hbm.at[pl.ds(wid * _CAP, _CAP)], bl1)
        pltpu.sync_copy(cnt_hbm.at[pl.ds(wid * _L, _L)], cnt_v)
        cnt = cnt_v[pl.ds(0, _L)][0]
        iota = lax.iota(jnp.int32, _L)
        cntv = lax.broadcast_in_dim(cnt, (_L,), ())

        def transpose(i, _):
            pos = i * _L + iota
            for c in range(_D):
                cv = lax.broadcast_in_dim(c, (_L,), ())
                vals = vfm[c, pl.ds(i * _L, _L)]
                plsc.store_scatter(vbm, [pos, cv], vals)
            return 0

        tr_trips = lax.div(cnt + _L - 1, _L)
        lax.fori_loop(0, tr_trips, transpose, 0)
        for row in range(_CAP // 128):
            for c16 in range(128 // _L):
                bl2[row, pl.ds(c16 * _L, _L)] = bl1[
                    pl.ds(row * 128 + c16 * _L, _L)
                ]
        for row in range(_CAP // 128):
            @pl.when(row * 128 < cnt)
            def _scatter(row=row):
                pltpu.sync_copy(
                    vbm.at[pl.ds(row * 128, 128), :],
                    out_hbm.at[bl2.at[row]],
                )

    return k2(vals1d, bpos1d, cnts)


def kernel(point_id, cffs_scaled, cff_scales):
    B = point_id.shape[0]
    table_t = cffs_scaled.T
    idx1d = point_id.astype(jnp.int32)
    vals1d, bpos1d, cnts = _sweep_kernel(idx1d, table_t, cff_scales, B)
    out = _permute_kernel(vals1d, bpos1d, cnts, B)
    return out[:B, : cffs_scaled.shape[1]]


# speedup vs baseline: 1.6446x; 1.1283x over previous
import functools

import jax
import jax.numpy as jnp
from jax import lax
from jax.experimental import pallas as pl
from jax.experimental.pallas import tpu as pltpu
from jax.experimental.pallas import tpu_sc as plsc

_NUM_WORKERS = 32
_TW = 128
_W_COLS = 244 * _TW
_C_COLS = 61 * _TW
_TAIL0 = _NUM_WORKERS * _W_COLS
_TAILW = 5 * _TW
_CAP = 2048
_L = 16
_D = 8


def kernel(point_id, cffs_scaled, cff_scales):
    B = point_id.shape[0]
    table_t = cffs_scaled.T
    idx1d = point_id.astype(jnp.int32)

    mesh = plsc.VectorSubcoreMesh(core_axis_name="c", subcore_axis_name="s")

    @functools.partial(
        pl.kernel,
        mesh=mesh,
        compiler_params=pltpu.CompilerParams(needs_layout_passes=False),
        out_type=jax.ShapeDtypeStruct((B + 128, _TW), jnp.float32),
        scratch_types=[
            pltpu.VMEM((B,), jnp.int32),
            pltpu.VMEM((_D, _C_COLS), jnp.float32),
            pltpu.VMEM((_CAP,), jnp.int32),
            pltpu.VMEM((_CAP,), jnp.int32),
            pltpu.VMEM((_CAP // 128, 128), jnp.int32),
            pltpu.VMEM((_D, _CAP), jnp.float32),
            pltpu.VMEM((128, _TW), jnp.float32),
            pltpu.VMEM((_L,), jnp.float32),
            pltpu.SemaphoreType.DMA,
        ],
    )
    def k(idx_hbm, table_hbm, scales_hbm, out_hbm, idx_vm, buf, rlist,
          blist, bl2, vfm, vbm, sc_v, sem):
        wid = lax.axis_index("s") * 2 + lax.axis_index("c")
        pltpu.sync_copy(idx_hbm, idx_vm)
        pltpu.sync_copy(scales_hbm, sc_v.at[pl.ds(0, _D)])
        iota = lax.iota(jnp.int32, _L)
        lo = wid * _W_COLS
        lov = lax.broadcast_in_dim(lo, (_L,), ())
        hiv = lov + _W_COLS
        wz = lax.broadcast_in_dim(wid == 0, (_L,), ())

        def match(i, cnt):
            v = idx_vm[pl.ds(i * _L, _L)]
            m = jnp.logical_and(v >= lov, v < hiv)
            m = jnp.logical_or(m, jnp.logical_and(wz, v >= _TAIL0))
            plsc.store_compressed(rlist.at[pl.ds(cnt, _L)], v, mask=m)
            plsc.store_compressed(
                blist.at[pl.ds(cnt, _L)], i * _L + iota, mask=m
            )
            pc = plsc.all_reduce_population_count(m)
            return lax.min(cnt + pc[0], _CAP - _L)

        cnt = lax.fori_loop(0, B // _L, match, 0)
        trips = lax.div(cnt + _L - 1, _L)
        cntv = lax.broadcast_in_dim(cnt, (_L,), ())

        s = sc_v[...]
        sfeat = [lax.broadcast_in_dim(s[c], (_L,), ()) for c in range(_D)]

        def process_chunk(col0, width):
            off = pl.multiple_of(col0, _TW)
            copies = [
                pltpu.async_copy(
                    table_hbm.at[:, pl.ds(off + t * _TW, _TW)],
                    buf.at[:, pl.ds(t * _TW, _TW)],
                    sem,
                )
                for t in range(width // _TW)
            ]
            for cp in copies:
                cp.wait()
            col0v = lax.broadcast_in_dim(col0, (_L,), ())

            def scan(j, _):
                pos = j * _L + iota
                v = rlist[pl.ds(j * _L, _L)]
                m = jnp.logical_and(pos < cntv, v >= col0v)
                m = jnp.logical_and(m, v < col0v + width)
                local = v - col0v
                for c in range(_D):
                    cv = lax.broadcast_in_dim(c, (_L,), ())
                    vals = plsc.load_gather(buf, [cv, local], mask=m)
                    plsc.store_scatter(
                        vfm, [cv, pos], vals * sfeat[c], mask=m
                    )
                return 0

            lax.fori_loop(0, trips, scan, 0)

        for ch in range(_W_COLS // _C_COLS):
            process_chunk(lo + ch * _C_COLS, _C_COLS)
        process_chunk(lo * 0 + _TAIL0, _TAILW)

        dummyv = (
            lax.broadcast_in_dim(B + wid * 4, (_L,), ())
            + lax.bitwise_and(iota, 3)
        )

        def pad(i, _):
            pos = i * _L + iota
            keep = pos < cntv
            cur = blist[pl.ds(i * _L, _L)]
            blist[pl.ds(i * _L, _L)] = jnp.where(keep, cur, dummyv)
            return 0

        lax.fori_loop(0, _CAP // _L, pad, 0)
        for row in range(_CAP // 128):
            for c16 in range(128 // _L):
                bl2[row, pl.ds(c16 * _L, _L)] = blist[
                    pl.ds(row * 128 + c16 * _L, _L)
                ]

        for row in range(_CAP // 128):
            @pl.when(row * 128 < cnt)
            def _scatter(row=row):
                def repack(i, _):
                    pos = row * 128 + i * _L + iota
                    lpos = i * _L + iota
                    for c in range(_D):
                        cv = lax.broadcast_in_dim(c, (_L,), ())
                        vals = plsc.load_gather(vfm, [cv, pos])
                        plsc.store_scatter(vbm, [lpos, cv], vals)
                    return 0

                lax.fori_loop(0, 128 // _L, repack, 0)
                pltpu.sync_copy(vbm, out_hbm.at[bl2.at[row]])

    out = k(idx1d, table_t, cff_scales)
    return out[:B, :_D]


# speedup vs baseline: 2.5881x; 1.5737x over previous
import functools

import jax
import jax.numpy as jnp
from jax import lax
from jax.experimental import pallas as pl
from jax.experimental.pallas import tpu as pltpu
from jax.experimental.pallas import tpu_sc as plsc

_NUM_WORKERS = 32
_TW = 128
_CH = 32


def kernel(point_id, cffs_scaled, cff_scales):
    B = point_id.shape[0]
    D = cffs_scaled.shape[1]
    L = 16
    b_per_w = B // _NUM_WORKERS

    table_t = cffs_scaled.T
    idx2d = point_id.astype(jnp.int32).reshape(_NUM_WORKERS, b_per_w)

    mesh = plsc.VectorSubcoreMesh(core_axis_name="c", subcore_axis_name="s")

    @functools.partial(
        pl.kernel,
        mesh=mesh,
        compiler_params=pltpu.CompilerParams(needs_layout_passes=False),
        out_type=jax.ShapeDtypeStruct((D, B), jnp.float32),
        scratch_types=[
            pltpu.VMEM((1, b_per_w), jnp.int32),
            pltpu.VMEM((_CH * D, _TW), jnp.float32),
            pltpu.VMEM((_CH * D, _TW), jnp.float32),
            pltpu.VMEM((D, b_per_w), jnp.float32),
            pltpu.VMEM((L,), jnp.float32),
            pltpu.SemaphoreType.DMA,
            pltpu.SemaphoreType.DMA,
        ],
    )
    def k(idx_hbm, table_hbm, scales_hbm, out_hbm, idx_vm, blk_a, blk_b,
          f_v, sc_v, sem_a, sem_b):
        wid = lax.axis_index("s") * 2 + lax.axis_index("c")
        pltpu.sync_copy(idx_hbm.at[pl.ds(wid, 1)], idx_vm)
        pltpu.sync_copy(scales_hbm, sc_v.at[pl.ds(0, D)])

        s = sc_v[...]
        iota = lax.iota(jnp.int32, L)
        n_chunks = b_per_w // _CH

        def fire(g, blk, sem):
            base = g * _CH
            copies = []
            for v16 in range(_CH // L):
                vec = idx_vm[0, pl.ds(base + v16 * L, L)]
                gbase = lax.shift_left(
                    lax.shift_right_logical(vec, 7), 7
                )
                for j in range(L):
                    off = pl.multiple_of(gbase[j], _TW)
                    copies.append(
                        pltpu.async_copy(
                            table_hbm.at[:, pl.ds(off, _TW)],
                            blk.at[pl.ds((v16 * L + j) * D, D), :],
                            sem,
                        )
                    )
            return copies

        def extract(g, blk):
            base = g * _CH
            for v16 in range(_CH // L):
                idx16 = idx_vm[0, pl.ds(base + v16 * L, L)]
                lanes = lax.bitwise_and(idx16, _TW - 1)
                rows0 = (v16 * L + iota) * D
                for c in range(D):
                    vals = plsc.load_gather(blk, [rows0 + c, lanes])
                    f_v[c, pl.ds(base + v16 * L, L)] = (
                        vals * lax.broadcast_in_dim(s[c], (L,), ())
                    )

        def pair(h, _):
            ga = h * 2
            gb = h * 2 + 1
            ca = fire(ga, blk_a, sem_a)
            cb = fire(gb, blk_b, sem_b)
            for cp in ca:
                cp.wait()
            extract(ga, blk_a)
            for cp in cb:
                cp.wait()
            extract(gb, blk_b)
            return 0

        lax.fori_loop(0, n_chunks // 2, pair, 0)
        pltpu.sync_copy(f_v, out_hbm.at[:, pl.ds(wid * b_per_w, b_per_w)])

    out = k(idx2d, table_t, cff_scales)
    return out.T


# speedup vs baseline: 2.6026x; 1.0056x over previous
import functools

import jax
import jax.numpy as jnp
from jax import lax
from jax.experimental import pallas as pl
from jax.experimental.pallas import tpu as pltpu
from jax.experimental.pallas import tpu_sc as plsc

_NUM_WORKERS = 32
_TW = 128
_CH = 32


def kernel(point_id, cffs_scaled, cff_scales):
    B = point_id.shape[0]
    D = cffs_scaled.shape[1]
    L = 16
    b_per_w = B // _NUM_WORKERS

    table_t = cffs_scaled.T
    idx2d = point_id.astype(jnp.int32).reshape(_NUM_WORKERS, b_per_w)

    mesh = plsc.VectorSubcoreMesh(core_axis_name="c", subcore_axis_name="s")

    @functools.partial(
        pl.kernel,
        mesh=mesh,
        compiler_params=pltpu.CompilerParams(
            needs_layout_passes=False, skip_device_barrier=True
        ),
        out_type=jax.ShapeDtypeStruct((D, B), jnp.float32),
        scratch_types=[
            pltpu.VMEM((1, b_per_w), jnp.int32),
            pltpu.VMEM((_CH * D, _TW), jnp.float32),
            pltpu.VMEM((_CH * D, _TW), jnp.float32),
            pltpu.VMEM((D, b_per_w), jnp.float32),
            pltpu.VMEM((L,), jnp.float32),
            pltpu.SemaphoreType.DMA,
            pltpu.SemaphoreType.DMA,
        ],
    )
    def k(idx_hbm, table_hbm, scales_hbm, out_hbm, idx_vm, blk_a, blk_b,
          f_v, sc_v, sem_a, sem_b):
        wid = lax.axis_index("s") * 2 + lax.axis_index("c")
        pltpu.sync_copy(idx_hbm.at[pl.ds(wid, 1)], idx_vm)
        pltpu.sync_copy(scales_hbm, sc_v.at[pl.ds(0, D)])

        s = sc_v[...]
        iota = lax.iota(jnp.int32, L)
        n_chunks = b_per_w // _CH

        def fire(g, blk, sem):
            base = g * _CH
            copies = []
            for v16 in range(_CH // L):
                vec = idx_vm[0, pl.ds(base + v16 * L, L)]
                gbase = lax.shift_left(
                    lax.shift_right_logical(vec, 7), 7
                )
                for j in range(L):
                    off = pl.multiple_of(gbase[j], _TW)
                    copies.append(
                        pltpu.async_copy(
                            table_hbm.at[:, pl.ds(off, _TW)],
                            blk.at[pl.ds((v16 * L + j) * D, D), :],
                            sem,
                        )
                    )
            return copies

        def extract(g, blk):
            base = g * _CH
            for v16 in range(_CH // L):
                idx16 = idx_vm[0, pl.ds(base + v16 * L, L)]
                lanes = lax.bitwise_and(idx16, _TW - 1)
                rows0 = (v16 * L + iota) * D
                for c in range(D):
                    vals = plsc.load_gather(blk, [rows0 + c, lanes])
                    f_v[c, pl.ds(base + v16 * L, L)] = (
                        vals * lax.broadcast_in_dim(s[c], (L,), ())
                    )

        def pair(h, _):
            ga = h * 2
            gb = h * 2 + 1
            ca = fire(ga, blk_a, sem_a)
            cb = fire(gb, blk_b, sem_b)
            for cp in ca:
                cp.wait()
            extract(ga, blk_a)
            for cp in cb:
                cp.wait()
            extract(gb, blk_b)
            return 0

        lax.fori_loop(0, n_chunks // 2, pair, 0)
        pltpu.sync_copy(f_v, out_hbm.at[:, pl.ds(wid * b_per_w, b_per_w)])

    out = k(idx2d, table_t, cff_scales)
    return out.T
